# Initial kernel scaffold; baseline (speedup 1.0000x reference)
import jax
import jax.numpy as jnp
from jax.experimental import pallas as pl


def kernel(x, edge_index, W1, b1, W2, b2, W3, b3, W4, b4):
    raise NotImplementedError("write your pallas kernel here")


# speedup vs baseline: 5.0537x; 5.0537x over previous
import functools

import jax
import jax.numpy as jnp
from jax import lax
from jax.experimental import pallas as pl
from jax.experimental.pallas import tpu as pltpu
from jax.experimental.pallas import tpu_sc as plsc

N = 100000
E = 1600000
NC, NS, L = 2, 16, 16
NW = NC * NS
EPW = E // NW
CH = 2000
NCH = EPW // CH
VPC = CH // L
SLAB = 8192
SLAB_SHIFT = 13
NSLAB = 13
NPAD = SLAB * NSLAB
BLK = 2048
NMM = 100352
ZR = N
EPALLOC = E + NW * NSLAB * 16 + 128
STG = 160
DEGPAD = SLAB + 128
ZBLK = SLAB // 128 // NS

_i32 = jnp.int32
_f32 = jnp.float32


@functools.cache
def _mesh():
    return plsc.VectorSubcoreMesh(core_axis_name="c", subcore_axis_name="s",
                                  num_cores=NC, num_subcores=NS)


def _lanes():
    return lax.iota(_i32, L)


def _count_body(dst_hbm, counts_hbm, dchunk, cmat):
    w = lax.axis_index("s") * NC + lax.axis_index("c")
    base = w * EPW

    def chunk(i, carry):
        pltpu.sync_copy(dst_hbm.at[pl.ds(pl.multiple_of(base + i * CH, 8), CH)], dchunk)

        def vreg(j, carry):
            sid = lax.shift_right_logical(dchunk[pl.ds(j * L, L)], SLAB_SHIFT)
            return tuple(carry[s] + jnp.where(sid == s, 1, 0)
                         for s in range(NSLAB))

        return lax.fori_loop(0, VPC, vreg, carry)

    carry = lax.fori_loop(0, NCH, chunk,
                          (jnp.zeros((L,), _i32),) * NSLAB)
    for s in range(NSLAB):
        cmat[pl.ds(s * L, L)] = carry[s]
    for s in range(NSLAB, 16):
        cmat[pl.ds(s * L, L)] = jnp.zeros((L,), _i32)
    pltpu.sync_copy(cmat, counts_hbm.at[w])


@functools.cache
def _a1():
    return pl.kernel(
        _count_body,
        out_type=jax.ShapeDtypeStruct((NW, 16 * L), _i32),
        mesh=_mesh(),
        scratch_types=[pltpu.VMEM((CH,), _i32), pltpu.VMEM((16 * L,), _i32)],
    )


def _dist_body(src_hbm, dst_hbm, woff_hbm, esrc, edstl,
               schunk, dchunk, tbufa, tbufb, gbuf, sent_s, sent_d, wbuf):
    base = (lax.axis_index("s") * NC + lax.axis_index("c")) * EPW
    pltpu.sync_copy(woff_hbm.at[lax.axis_index("s") * NC
                                + lax.axis_index("c")], wbuf)
    wo_vec = wbuf[...]
    lanes = _lanes()
    for k in range(1):
        sent_s[pl.ds(0, L)] = jnp.full((L,), ZR, _i32)
        sent_d[pl.ds(0, L)] = jnp.zeros((L,), _i32)

    def vreg(j, posv, store):
        sv = schunk[pl.ds(j * L, L)]
        dv = dchunk[pl.ds(j * L, L)]
        sid = lax.shift_right_logical(dv, SLAB_SHIFT)
        dloc = jnp.bitwise_and(dv, SLAB - 1)
        dchunk[pl.ds(j * L, L)] = dloc
        rank = jnp.zeros((L,), _i32)
        for dd in range(1, L):
            sh = sid.at[jnp.maximum(lanes - dd, 0)].get(
                mode="promise_in_bounds")
            rank = rank + jnp.where((lanes >= dd) & (sh == sid), 1, 0)
        tgt = (wo_vec + posv).at[sid].get(mode="promise_in_bounds") + rank
        store(tgt)
        for l in range(L):
            posv = posv + jnp.where(lanes == sid[l], 1, 0)
        return posv

    def chunk(i, posv):
        off = pl.multiple_of(base + i * CH, 8)
        pltpu.sync_copy(src_hbm.at[pl.ds(off, CH)], schunk)
        pltpu.sync_copy(dst_hbm.at[pl.ds(off, CH)], dchunk)

        def row(r, posv):
            for jj in range(8):
                def store(tgt, r=r, jj=jj):
                    tbufa[r, pl.ds(jj * L, L)] = tgt
                posv = vreg(r * 8 + jj, posv, store)
            return posv

        posv = lax.fori_loop(0, 15, row, posv)
        for jj in range(5):
            def store(tgt, jj=jj):
                tbufb[0, pl.ds(jj * L, L)] = tgt
            posv = vreg(120 + jj, posv, store)

        def dmarow(r, u):
            off_r = pl.multiple_of(r * 128, 8)
            pltpu.sync_copy(schunk.at[pl.ds(off_r, 128)],
                            esrc.at[tbufa.at[r]])
            pltpu.sync_copy(dchunk.at[pl.ds(off_r, 128)],
                            edstl.at[tbufa.at[r]])
            return u

        lax.fori_loop(0, 15, dmarow, jnp.zeros((), _i32))
        pltpu.sync_copy(schunk.at[pl.ds(1920, 80)], esrc.at[tbufb.at[0]])
        pltpu.sync_copy(dchunk.at[pl.ds(1920, 80)], edstl.at[tbufb.at[0]])
        return posv

    posv = lax.fori_loop(0, NCH, chunk, jnp.zeros((L,), _i32))

    for s in range(NSLAB):
        cnt = posv[s]
        cpad = lax.shift_left(lax.shift_right_logical(cnt + 15, 3), 3)
        wo_s = wo_vec[s]
        gbuf[0, pl.ds(0, L)] = wo_s + jnp.minimum(cnt + lanes, cpad - 1)
        pltpu.sync_copy(sent_s, esrc.at[gbuf.at[0]])
        pltpu.sync_copy(sent_d, edstl.at[gbuf.at[0]])


@functools.cache
def _a2():
    return pl.kernel(
        _dist_body,
        out_type=(jax.ShapeDtypeStruct((EPALLOC,), _i32),
                  jax.ShapeDtypeStruct((EPALLOC,), _i32)),
        mesh=_mesh(),
        scratch_types=[
            pltpu.VMEM((CH,), _i32), pltpu.VMEM((CH,), _i32),
            pltpu.VMEM((15, 128), _i32),
            pltpu.VMEM((1, 80), _i32),
            pltpu.VMEM((1, L), _i32),
            pltpu.VMEM((L,), _i32),
            pltpu.VMEM((L,), _i32),
            pltpu.VMEM((L,), _i32),
        ],
    )


def _deg_body(esrc, edstl, sb_hbm, deg_hbm,
              sbuf, dbuf, ones_v, zbuf, sbvec, acc1):
    c = lax.axis_index("c")
    t = lax.axis_index("s")
    lanes = _lanes()
    pltpu.sync_copy(sb_hbm, sbvec)
    sb = sbvec[...]

    def vfill(k, u):
        ones_v[pl.ds(k * L, L)] = jnp.ones((L,), _f32)
        zbuf[pl.ds(k * L, L)] = jnp.zeros((L,), _f32)
        return u

    lax.fori_loop(0, 128 // L, vfill, jnp.zeros((), _i32))

    for si in range(7):
        s = 2 * si + c

        def do_slab(s=s):
            nblkz = DEGPAD // 128

            def zblk(i, u):
                b = t + i * NS

                @pl.when(b < nblkz)
                def _():
                    pltpu.sync_copy(zbuf, acc1.at[pl.ds(pl.multiple_of(b * 128, 8), 128)])

                return u

            lax.fori_loop(0, (nblkz + NS - 1) // NS, zblk,
                          jnp.zeros((), _i32))
            plsc.subcore_barrier()

            rs = jnp.where(c == 0, sb[2 * si], sb[2 * si + 1])
            re = jnp.where(c == 0, sb[2 * si + 1], sb[2 * si + 2])
            nch = lax.shift_right_logical(re - rs + 127, 7)
            nt = jnp.maximum(nch - t + NS - 1, 0) // NS

            def chunk(i, u):
                c0 = rs + (t + i * NS) * 128
                pltpu.sync_copy(esrc.at[pl.ds(pl.multiple_of(c0, 8), 128)], sbuf)
                pltpu.sync_copy(edstl.at[pl.ds(pl.multiple_of(c0, 8), 128)], dbuf)
                for j in range(128 // L):
                    valid = (c0 + j * L + lanes) < re
                    sv = sbuf[pl.ds(j * L, L)]
                    dv = dbuf[pl.ds(j * L, L)]
                    keep = jnp.logical_and(valid, sv != ZR)
                    dbuf[pl.ds(j * L, L)] = jnp.where(keep, dv, SLAB)
                pltpu.sync_copy(ones_v, acc1.at[dbuf], add=True)
                return u

            lax.fori_loop(0, nt, chunk, jnp.zeros((), _i32))
            plsc.subcore_barrier()
            for k in range(ZBLK):
                b = t * ZBLK + k
                pltpu.sync_copy(acc1.at[pl.ds(pl.multiple_of(b * 128, 8), 128)],
                                deg_hbm.at[pl.ds(pl.multiple_of(s * SLAB + b * 128, 8), 128)])
            plsc.subcore_barrier()

        if si < 6:
            do_slab()
        else:
            pl.when(c == 0)(do_slab)


@functools.cache
def _deg():
    return pl.kernel(
        _deg_body,
        out_type=jax.ShapeDtypeStruct((NPAD,), _f32),
        mesh=_mesh(),
        scratch_types=[
            pltpu.VMEM((128,), _i32), pltpu.VMEM((128,), _i32),
            pltpu.VMEM((128,), _f32), pltpu.VMEM((128,), _f32),
            pltpu.VMEM((L,), _i32),
            pltpu.VMEM_SHARED((DEGPAD,), _f32),
        ],
    )


def _agg_body(d, hs, esrc, edstl, sb_hbm, agg_hbm,
              sbuf, dbuf, rows, zbuf, sbvec, sem, acc):
    c = lax.axis_index("c")
    t = lax.axis_index("s")
    lanes = _lanes()
    pltpu.sync_copy(sb_hbm, sbvec)
    sb = sbvec[...]

    def zrow(i, u):
        def zcol(j, u):
            zbuf[i, pl.ds(j * L, L)] = jnp.zeros((L,), _f32)
            return u

        return lax.fori_loop(0, d // L, zcol, u)

    lax.fori_loop(0, 128, zrow, jnp.zeros((), _i32))

    for si in range(7):
        s = 2 * si + c

        def do_slab(s=s):
            for k in range(ZBLK):
                b = t * ZBLK + k
                pltpu.sync_copy(zbuf, acc.at[pl.ds(pl.multiple_of(b * 128, 8), 128)])
            plsc.subcore_barrier()

            rs = jnp.where(c == 0, sb[2 * si], sb[2 * si + 1])
            re = jnp.where(c == 0, sb[2 * si + 1], sb[2 * si + 2])
            nch = lax.shift_right_logical(re - rs + 127, 7)
            nt = jnp.maximum(nch - t + NS - 1, 0) // NS

            def chunk(i, u):
                c0 = rs + (t + i * NS) * 128
                pltpu.sync_copy(esrc.at[pl.ds(pl.multiple_of(c0, 8), 128)], sbuf)
                pltpu.sync_copy(edstl.at[pl.ds(pl.multiple_of(c0, 8), 128)], dbuf)
                for j in range(128 // L):
                    valid = (c0 + j * L + lanes) < re
                    sv = sbuf[pl.ds(j * L, L)]
                    dv = dbuf[pl.ds(j * L, L)]
                    sbuf[pl.ds(j * L, L)] = jnp.where(valid, sv, ZR)
                    dbuf[pl.ds(j * L, L)] = jnp.where(valid, dv, 0)
                pltpu.async_copy(hs.at[sbuf], rows, sem).wait()
                pltpu.sync_copy(rows, acc.at[dbuf], add=True)
                return u

            lax.fori_loop(0, nt, chunk, jnp.zeros((), _i32))
            plsc.subcore_barrier()
            for k in range(ZBLK):
                b = t * ZBLK + k
                pltpu.sync_copy(acc.at[pl.ds(pl.multiple_of(b * 128, 8), 128)],
                                agg_hbm.at[pl.ds(pl.multiple_of(s * SLAB + b * 128, 8), 128)])
            plsc.subcore_barrier()

        if si < 6:
            do_slab()
        else:
            pl.when(c == 0)(do_slab)


@functools.cache
def _agg(d):
    return pl.kernel(
        functools.partial(_agg_body, d),
        out_type=jax.ShapeDtypeStruct((NPAD, d), _f32),
        mesh=_mesh(),
        scratch_types=[
            pltpu.VMEM((128,), _i32), pltpu.VMEM((128,), _i32),
            pltpu.VMEM((128, d), _f32), pltpu.VMEM((128, d), _f32),
            pltpu.VMEM((L,), _i32),
            pltpu.SemaphoreType.DMA,
            pltpu.VMEM_SHARED((SLAB, d), _f32),
        ],
    )


def _mm1_body(x_ref, w_ref, deg_ref, hs_ref, dinv_ref):
    dv = lax.rsqrt(deg_ref[...] + 1.0)
    hs_ref[...] = jnp.dot(x_ref[...], w_ref[...],
                          preferred_element_type=_f32) * dv
    dinv_ref[...] = dv


@functools.cache
def _mm1(din, dout):
    return pl.pallas_call(
        _mm1_body,
        grid=(NMM // BLK,),
        in_specs=[pl.BlockSpec((BLK, din), lambda i: (i, 0)),
                  pl.BlockSpec((din, dout), lambda i: (0, 0)),
                  pl.BlockSpec((BLK, 1), lambda i: (i, 0))],
        out_specs=[pl.BlockSpec((BLK, dout), lambda i: (i, 0)),
                   pl.BlockSpec((BLK, 1), lambda i: (i, 0))],
        out_shape=[jax.ShapeDtypeStruct((NMM, dout), _f32),
                   jax.ShapeDtypeStruct((NMM, 1), _f32)],
    )


def _mid_body(nout, agg_ref, hs_ref, dinv_ref, b_ref, w_ref, *out_refs):
    dv = dinv_ref[...]
    tv = jnp.maximum((agg_ref[...] + hs_ref[...]) * dv + b_ref[...], 0.0)
    w = w_ref[...]
    step = w.shape[1] // nout
    for k, o_ref in enumerate(out_refs):
        o_ref[...] = jnp.dot(tv, w[:, k * step:(k + 1) * step],
                             preferred_element_type=_f32) * dv


@functools.cache
def _mid(dprev, dnext, nout):
    step = dnext // nout
    return pl.pallas_call(
        functools.partial(_mid_body, nout),
        grid=(NMM // BLK,),
        in_specs=[pl.BlockSpec((BLK, dprev), lambda i: (i, 0)),
                  pl.BlockSpec((BLK, dprev), lambda i: (i, 0)),
                  pl.BlockSpec((BLK, 1), lambda i: (i, 0)),
                  pl.BlockSpec((1, dprev), lambda i: (0, 0)),
                  pl.BlockSpec((dprev, dnext), lambda i: (0, 0))],
        out_specs=[pl.BlockSpec((BLK, step), lambda i: (i, 0))] * nout,
        out_shape=[jax.ShapeDtypeStruct((NMM, step), _f32)] * nout,
    )


def _mm4_body(a1_ref, a2_ref, h1_ref, h2_ref, dinv_ref, b_ref, w_ref,
              out_ref):
    dv = dinv_ref[...]
    b = b_ref[...]
    t1 = jnp.maximum((a1_ref[...] + h1_ref[...]) * dv + b[:, :128], 0.0)
    t2 = jnp.maximum((a2_ref[...] + h2_ref[...]) * dv + b[:, 128:], 0.0)
    w = w_ref[...]
    out_ref[...] = (jnp.dot(t1, w[:128], preferred_element_type=_f32)
                    + jnp.dot(t2, w[128:], preferred_element_type=_f32)) * dv


@functools.cache
def _mm4():
    return pl.pallas_call(
        _mm4_body,
        grid=(NMM // BLK,),
        in_specs=[pl.BlockSpec((BLK, 128), lambda i: (i, 0)),
                  pl.BlockSpec((BLK, 128), lambda i: (i, 0)),
                  pl.BlockSpec((BLK, 128), lambda i: (i, 0)),
                  pl.BlockSpec((BLK, 128), lambda i: (i, 0)),
                  pl.BlockSpec((BLK, 1), lambda i: (i, 0)),
                  pl.BlockSpec((1, 256), lambda i: (0, 0)),
                  pl.BlockSpec((256, 128), lambda i: (0, 0))],
        out_specs=[pl.BlockSpec((BLK, 128), lambda i: (i, 0))],
        out_shape=[jax.ShapeDtypeStruct((NMM, 128), _f32)],
    )


def _final_body(agg_ref, hs_ref, dinv_ref, b_ref, out_ref):
    out_ref[...] = ((agg_ref[...] + hs_ref[...]) * dinv_ref[...]
                    + b_ref[...])


@functools.cache
def _final():
    return pl.pallas_call(
        _final_body,
        grid=(NMM // BLK,),
        in_specs=[pl.BlockSpec((BLK, 128), lambda i: (i, 0)),
                  pl.BlockSpec((BLK, 128), lambda i: (i, 0)),
                  pl.BlockSpec((BLK, 1), lambda i: (i, 0)),
                  pl.BlockSpec((1, 128), lambda i: (0, 0))],
        out_specs=pl.BlockSpec((BLK, 128), lambda i: (i, 0)),
        out_shape=jax.ShapeDtypeStruct((NMM, 128), _f32),
    )


def kernel(x, edge_index, W1, b1, W2, b2, W3, b3, W4, b4):
    src = edge_index[0].astype(_i32)
    dst = edge_index[1].astype(_i32)

    counts = _a1()(dst).reshape(NW, 16, L)[:, :NSLAB, :].sum(-1)
    cpad = ((counts + 15) // 8) * 8
    tot = cpad.sum(axis=0)
    ss = jnp.concatenate([jnp.zeros((1,), _i32),
                          jnp.cumsum(tot).astype(_i32)])
    woff = ss[None, :NSLAB] + (jnp.cumsum(cpad, axis=0) - cpad)
    woff16 = jnp.zeros((NW, L), _i32).at[:, :NSLAB].set(woff)
    sb16 = jnp.zeros((L,), _i32).at[:NSLAB + 1].set(ss)

    esrc, edstl = _a2()(src, dst, woff16)
    deg = _deg()(esrc, edstl, sb16)

    xp = jnp.zeros((NMM, x.shape[1]), _f32).at[:N].set(x)
    W1p = jnp.zeros((x.shape[1], 128), _f32).at[:, :64].set(W1)
    b1p = jnp.zeros((128,), _f32).at[:64].set(b1)
    W2p = jnp.zeros((128, 128), _f32).at[:64].set(W2)
    hs1, dinv = _mm1(x.shape[1], 128)(xp, W1p, deg[:NMM, None])
    agg1 = _agg(128)(hs1, esrc, edstl, sb16)
    hs2, = _mid(128, 128, 1)(agg1[:NMM], hs1, dinv, b1p[None], W2p)
    agg2 = _agg(128)(hs2, esrc, edstl, sb16)
    hs3a, hs3b = _mid(128, 256, 2)(agg2[:NMM], hs2, dinv, b2[None], W3)
    agg3a = _agg(128)(hs3a, esrc, edstl, sb16)
    agg3b = _agg(128)(hs3b, esrc, edstl, sb16)
    W4p = jnp.zeros((256, 128), _f32).at[:, :3].set(W4)
    b4p = jnp.zeros((128,), _f32).at[:3].set(b4)
    hs4, = _mm4()(agg3a[:NMM], agg3b[:NMM], hs3a, hs3b, dinv, b3[None], W4p)
    agg4 = _agg(128)(hs4, esrc, edstl, sb16)
    outp = _final()(agg4[:NMM], hs4, dinv, b4p[None])
    return outp[:N, :3]


# speedup vs baseline: 6.6411x; 1.3141x over previous
import functools

import jax
import jax.numpy as jnp
from jax import lax
from jax.experimental import pallas as pl
from jax.experimental.pallas import tpu as pltpu
from jax.experimental.pallas import tpu_sc as plsc

N = 100000
E = 1600000
NC, NS, L = 2, 16, 16
NW = NC * NS
EPW = E // NW
CH = 2000
NCH = EPW // CH
VPC = CH // L
SLAB = 8192
SLAB_SHIFT = 13
NSLAB = 13
NPAD = SLAB * NSLAB
BLK = 2048
NMM = 100352
ZR = N
EPALLOC = E + NW * NSLAB * 16 + 128
STG = 160
DEGPAD = SLAB + 128
ZBLK = SLAB // 128 // NS

_i32 = jnp.int32
_f32 = jnp.float32


@functools.cache
def _mesh():
    return plsc.VectorSubcoreMesh(core_axis_name="c", subcore_axis_name="s",
                                  num_cores=NC, num_subcores=NS)


def _lanes():
    return lax.iota(_i32, L)


def _count_body(dst_hbm, counts_hbm, dchunk, cmat):
    w = lax.axis_index("s") * NC + lax.axis_index("c")
    base = w * EPW

    def chunk(i, carry):
        pltpu.sync_copy(dst_hbm.at[pl.ds(pl.multiple_of(base + i * CH, 8), CH)], dchunk)

        def vreg(j, carry):
            sid = lax.shift_right_logical(dchunk[pl.ds(j * L, L)], SLAB_SHIFT)
            return tuple(carry[s] + jnp.where(sid == s, 1, 0)
                         for s in range(NSLAB))

        return lax.fori_loop(0, VPC, vreg, carry)

    carry = lax.fori_loop(0, NCH, chunk,
                          (jnp.zeros((L,), _i32),) * NSLAB)
    for s in range(NSLAB):
        cmat[pl.ds(s * L, L)] = carry[s]
    for s in range(NSLAB, 16):
        cmat[pl.ds(s * L, L)] = jnp.zeros((L,), _i32)
    pltpu.sync_copy(cmat, counts_hbm.at[w])


@functools.cache
def _a1():
    return pl.kernel(
        _count_body,
        out_type=jax.ShapeDtypeStruct((NW, 16 * L), _i32),
        mesh=_mesh(),
        scratch_types=[pltpu.VMEM((CH,), _i32), pltpu.VMEM((16 * L,), _i32)],
    )


def _dist_body(src_hbm, dst_hbm, woff_hbm, esrc, edstl,
               schunk, dchunk, tbufa, tbufb, gbuf, sent_s, sent_d, wbuf):
    base = (lax.axis_index("s") * NC + lax.axis_index("c")) * EPW
    pltpu.sync_copy(woff_hbm.at[lax.axis_index("s") * NC
                                + lax.axis_index("c")], wbuf)
    wo_vec = wbuf[...]
    lanes = _lanes()
    for k in range(1):
        sent_s[pl.ds(0, L)] = jnp.full((L,), ZR, _i32)
        sent_d[pl.ds(0, L)] = jnp.zeros((L,), _i32)

    def vreg(j, posv, store):
        sv = schunk[pl.ds(j * L, L)]
        dv = dchunk[pl.ds(j * L, L)]
        sid = lax.shift_right_logical(dv, SLAB_SHIFT)
        dloc = jnp.bitwise_and(dv, SLAB - 1)
        dchunk[pl.ds(j * L, L)] = dloc
        rank = jnp.zeros((L,), _i32)
        for dd in range(1, L):
            sh = sid.at[jnp.maximum(lanes - dd, 0)].get(
                mode="promise_in_bounds")
            rank = rank + jnp.where((lanes >= dd) & (sh == sid), 1, 0)
        tgt = (wo_vec + posv).at[sid].get(mode="promise_in_bounds") + rank
        store(tgt)
        for l in range(L):
            posv = posv + jnp.where(lanes == sid[l], 1, 0)
        return posv

    def chunk(i, posv):
        off = pl.multiple_of(base + i * CH, 8)
        pltpu.sync_copy(src_hbm.at[pl.ds(off, CH)], schunk)
        pltpu.sync_copy(dst_hbm.at[pl.ds(off, CH)], dchunk)

        def row(r, posv):
            for jj in range(8):
                def store(tgt, r=r, jj=jj):
                    tbufa[r, pl.ds(jj * L, L)] = tgt
                posv = vreg(r * 8 + jj, posv, store)
            return posv

        posv = lax.fori_loop(0, 15, row, posv)
        for jj in range(5):
            def store(tgt, jj=jj):
                tbufb[0, pl.ds(jj * L, L)] = tgt
            posv = vreg(120 + jj, posv, store)

        def dmarow(r, u):
            off_r = pl.multiple_of(r * 128, 8)
            pltpu.sync_copy(schunk.at[pl.ds(off_r, 128)],
                            esrc.at[tbufa.at[r]])
            pltpu.sync_copy(dchunk.at[pl.ds(off_r, 128)],
                            edstl.at[tbufa.at[r]])
            return u

        lax.fori_loop(0, 15, dmarow, jnp.zeros((), _i32))
        pltpu.sync_copy(schunk.at[pl.ds(1920, 80)], esrc.at[tbufb.at[0]])
        pltpu.sync_copy(dchunk.at[pl.ds(1920, 80)], edstl.at[tbufb.at[0]])
        return posv

    posv = lax.fori_loop(0, NCH, chunk, jnp.zeros((L,), _i32))

    for s in range(NSLAB):
        cnt = posv[s]
        cpad = lax.shift_left(lax.shift_right_logical(cnt + 15, 3), 3)
        wo_s = wo_vec[s]
        gbuf[0, pl.ds(0, L)] = wo_s + jnp.minimum(cnt + lanes, cpad - 1)
        pltpu.sync_copy(sent_s, esrc.at[gbuf.at[0]])
        pltpu.sync_copy(sent_d, edstl.at[gbuf.at[0]])


@functools.cache
def _a2():
    return pl.kernel(
        _dist_body,
        out_type=(jax.ShapeDtypeStruct((EPALLOC,), _i32),
                  jax.ShapeDtypeStruct((EPALLOC,), _i32)),
        mesh=_mesh(),
        scratch_types=[
            pltpu.VMEM((CH,), _i32), pltpu.VMEM((CH,), _i32),
            pltpu.VMEM((15, 128), _i32),
            pltpu.VMEM((1, 80), _i32),
            pltpu.VMEM((1, L), _i32),
            pltpu.VMEM((L,), _i32),
            pltpu.VMEM((L,), _i32),
            pltpu.VMEM((L,), _i32),
        ],
    )


def _deg_body(esrc, edstl, sb_hbm, deg_hbm,
              sbuf, dbuf, ones_v, zbuf, sbvec, acc1):
    c = lax.axis_index("c")
    t = lax.axis_index("s")
    lanes = _lanes()
    pltpu.sync_copy(sb_hbm, sbvec)
    sb = sbvec[...]

    def vfill(k, u):
        ones_v[pl.ds(k * L, L)] = jnp.ones((L,), _f32)
        zbuf[pl.ds(k * L, L)] = jnp.zeros((L,), _f32)
        return u

    lax.fori_loop(0, 128 // L, vfill, jnp.zeros((), _i32))

    for si in range(7):
        s = 2 * si + c

        def do_slab(s=s):
            nblkz = DEGPAD // 128

            def zblk(i, u):
                b = t + i * NS

                @pl.when(b < nblkz)
                def _():
                    pltpu.sync_copy(zbuf, acc1.at[pl.ds(pl.multiple_of(b * 128, 8), 128)])

                return u

            lax.fori_loop(0, (nblkz + NS - 1) // NS, zblk,
                          jnp.zeros((), _i32))
            plsc.subcore_barrier()

            rs = jnp.where(c == 0, sb[2 * si], sb[2 * si + 1])
            re = jnp.where(c == 0, sb[2 * si + 1], sb[2 * si + 2])
            nch = lax.shift_right_logical(re - rs + 127, 7)
            nt = jnp.maximum(nch - t + NS - 1, 0) // NS

            def chunk(i, u):
                c0 = rs + (t + i * NS) * 128
                pltpu.sync_copy(esrc.at[pl.ds(pl.multiple_of(c0, 8), 128)], sbuf)
                pltpu.sync_copy(edstl.at[pl.ds(pl.multiple_of(c0, 8), 128)], dbuf)
                for j in range(128 // L):
                    valid = (c0 + j * L + lanes) < re
                    sv = sbuf[pl.ds(j * L, L)]
                    dv = dbuf[pl.ds(j * L, L)]
                    keep = jnp.logical_and(valid, sv != ZR)
                    dbuf[pl.ds(j * L, L)] = jnp.where(keep, dv, SLAB)
                pltpu.sync_copy(ones_v, acc1.at[dbuf], add=True)
                return u

            lax.fori_loop(0, nt, chunk, jnp.zeros((), _i32))
            plsc.subcore_barrier()
            for k in range(ZBLK):
                b = t * ZBLK + k
                pltpu.sync_copy(acc1.at[pl.ds(pl.multiple_of(b * 128, 8), 128)],
                                deg_hbm.at[pl.ds(pl.multiple_of(s * SLAB + b * 128, 8), 128)])
            plsc.subcore_barrier()

        if si < 6:
            do_slab()
        else:
            pl.when(c == 0)(do_slab)


@functools.cache
def _deg():
    return pl.kernel(
        _deg_body,
        out_type=jax.ShapeDtypeStruct((NPAD,), _f32),
        mesh=_mesh(),
        scratch_types=[
            pltpu.VMEM((128,), _i32), pltpu.VMEM((128,), _i32),
            pltpu.VMEM((128,), _f32), pltpu.VMEM((128,), _f32),
            pltpu.VMEM((L,), _i32),
            pltpu.VMEM_SHARED((DEGPAD,), _f32),
        ],
    )


def _agg_body(d, hs, esrc, edstl, sb_hbm, agg_hbm,
              sbuf0, sbuf1, dbuf0, dbuf1, rows0, rows1, zbuf, sbvec,
              ld0, ld1, g0, g1, sc0, sc1, acc):
    c = lax.axis_index("c")
    t = lax.axis_index("s")
    lanes = _lanes()
    sbufs, dbufs = (sbuf0, sbuf1), (dbuf0, dbuf1)
    rows, lds, gs, scs = (rows0, rows1), (ld0, ld1), (g0, g1), (sc0, sc1)
    pltpu.sync_copy(sb_hbm, sbvec)
    sb = sbvec[...]

    def zrow(i, u):
        def zcol(j, u):
            zbuf[i, pl.ds(j * L, L)] = jnp.zeros((L,), _f32)
            return u

        return lax.fori_loop(0, d // L, zcol, u)

    lax.fori_loop(0, 128, zrow, jnp.zeros((), _i32))

    for si in range(7):
        s = 2 * si + c

        def do_slab(s=s, si=si):
            for k in range(ZBLK):
                b = t * ZBLK + k
                pltpu.sync_copy(zbuf, acc.at[pl.ds(pl.multiple_of(b * 128, 8), 128)])
            plsc.subcore_barrier()

            rs = jnp.where(c == 0, sb[2 * si], sb[2 * si + 1])
            re = jnp.where(c == 0, sb[2 * si + 1], sb[2 * si + 2])
            nch = lax.shift_right_logical(re - rs + 127, 7)
            nt = jnp.maximum(nch - t + NS - 1, 0) // NS

            def issue_loads(k, b):
                c0 = pl.multiple_of(rs + (t + k * NS) * 128, 8)
                pltpu.async_copy(esrc.at[pl.ds(c0, 128)], sbufs[b], lds[b])
                pltpu.async_copy(edstl.at[pl.ds(c0, 128)], dbufs[b], lds[b])

            @pl.when(nt > 0)
            def _():
                issue_loads(jnp.zeros((), _i32), 0)

            def step(i2, u):
                for b in (0, 1):
                    k = 2 * i2 + b

                    @pl.when(k < nt)
                    def _(b=b, k=k):
                        pltpu.make_async_copy(
                            esrc.at[pl.ds(0, 128)], sbufs[b], lds[b]).wait()
                        pltpu.make_async_copy(
                            esrc.at[pl.ds(0, 128)], dbufs[b], lds[b]).wait()
                        c0 = rs + (t + k * NS) * 128
                        for j in range(128 // L):
                            valid = (c0 + j * L + lanes) < re
                            sv = sbufs[b][pl.ds(j * L, L)]
                            dv = dbufs[b][pl.ds(j * L, L)]
                            sbufs[b][pl.ds(j * L, L)] = jnp.where(valid, sv, ZR)
                            dbufs[b][pl.ds(j * L, L)] = jnp.where(valid, dv, 0)

                        @pl.when(k + 1 < nt)
                        def _():
                            issue_loads(k + 1, b ^ 1)

                        @pl.when(k >= 2)
                        def _():
                            pltpu.make_async_copy(
                                rows[b], acc.at[dbufs[b]], scs[b]).wait()

                        pltpu.async_copy(hs.at[sbufs[b]], rows[b], gs[b])
                        pltpu.make_async_copy(hs.at[sbufs[b]], rows[b],
                                              gs[b]).wait()
                        pltpu.async_copy(rows[b], acc.at[dbufs[b]], scs[b],
                                         add=True)
                return u

            lax.fori_loop(0, (nt + 1) // 2, step, jnp.zeros((), _i32))
            for b in (0, 1):
                @pl.when(nt > b)
                def _(b=b):
                    pltpu.make_async_copy(rows[b], acc.at[dbufs[b]],
                                          scs[b]).wait()
            plsc.subcore_barrier()
            for k in range(ZBLK):
                b = t * ZBLK + k
                pltpu.sync_copy(acc.at[pl.ds(pl.multiple_of(b * 128, 8), 128)],
                                agg_hbm.at[pl.ds(pl.multiple_of(s * SLAB + b * 128, 8), 128)])
            plsc.subcore_barrier()

        if si < 6:
            do_slab()
        else:
            pl.when(c == 0)(do_slab)


@functools.cache
def _agg(d):
    return pl.kernel(
        functools.partial(_agg_body, d),
        out_type=jax.ShapeDtypeStruct((NPAD, d), _f32),
        mesh=_mesh(),
        scratch_types=[
            pltpu.VMEM((128,), _i32), pltpu.VMEM((128,), _i32),
            pltpu.VMEM((128,), _i32), pltpu.VMEM((128,), _i32),
            pltpu.VMEM((128, d), _f32), pltpu.VMEM((128, d), _f32),
            pltpu.VMEM((128, d), _f32),
            pltpu.VMEM((L,), _i32),
            pltpu.SemaphoreType.DMA, pltpu.SemaphoreType.DMA,
            pltpu.SemaphoreType.DMA, pltpu.SemaphoreType.DMA,
            pltpu.SemaphoreType.DMA, pltpu.SemaphoreType.DMA,
            pltpu.VMEM_SHARED((SLAB, d), _f32),
        ],
    )


def _mm1_body(x_ref, w_ref, deg_ref, hs_ref, dinv_ref):
    dv = lax.rsqrt(deg_ref[...] + 1.0)
    hs_ref[...] = jnp.dot(x_ref[...], w_ref[...],
                          preferred_element_type=_f32) * dv
    dinv_ref[...] = dv


@functools.cache
def _mm1(din, dout):
    return pl.pallas_call(
        _mm1_body,
        grid=(NMM // BLK,),
        in_specs=[pl.BlockSpec((BLK, din), lambda i: (i, 0)),
                  pl.BlockSpec((din, dout), lambda i: (0, 0)),
                  pl.BlockSpec((BLK, 1), lambda i: (i, 0))],
        out_specs=[pl.BlockSpec((BLK, dout), lambda i: (i, 0)),
                   pl.BlockSpec((BLK, 1), lambda i: (i, 0))],
        out_shape=[jax.ShapeDtypeStruct((NMM, dout), _f32),
                   jax.ShapeDtypeStruct((NMM, 1), _f32)],
    )


def _mid_body(nout, agg_ref, hs_ref, dinv_ref, b_ref, w_ref, *out_refs):
    dv = dinv_ref[...]
    tv = jnp.maximum((agg_ref[...] + hs_ref[...]) * dv + b_ref[...], 0.0)
    w = w_ref[...]
    step = w.shape[1] // nout
    for k, o_ref in enumerate(out_refs):
        o_ref[...] = jnp.dot(tv, w[:, k * step:(k + 1) * step],
                             preferred_element_type=_f32) * dv


@functools.cache
def _mid(dprev, dnext, nout):
    step = dnext // nout
    return pl.pallas_call(
        functools.partial(_mid_body, nout),
        grid=(NMM // BLK,),
        in_specs=[pl.BlockSpec((BLK, dprev), lambda i: (i, 0)),
                  pl.BlockSpec((BLK, dprev), lambda i: (i, 0)),
                  pl.BlockSpec((BLK, 1), lambda i: (i, 0)),
                  pl.BlockSpec((1, dprev), lambda i: (0, 0)),
                  pl.BlockSpec((dprev, dnext), lambda i: (0, 0))],
        out_specs=[pl.BlockSpec((BLK, step), lambda i: (i, 0))] * nout,
        out_shape=[jax.ShapeDtypeStruct((NMM, step), _f32)] * nout,
    )


def _mm4_body(a1_ref, a2_ref, h1_ref, h2_ref, dinv_ref, b_ref, w_ref,
              out_ref):
    dv = dinv_ref[...]
    b = b_ref[...]
    t1 = jnp.maximum((a1_ref[...] + h1_ref[...]) * dv + b[:, :128], 0.0)
    t2 = jnp.maximum((a2_ref[...] + h2_ref[...]) * dv + b[:, 128:], 0.0)
    w = w_ref[...]
    out_ref[...] = (jnp.dot(t1, w[:128], preferred_element_type=_f32)
                    + jnp.dot(t2, w[128:], preferred_element_type=_f32)) * dv


@functools.cache
def _mm4():
    return pl.pallas_call(
        _mm4_body,
        grid=(NMM // BLK,),
        in_specs=[pl.BlockSpec((BLK, 128), lambda i: (i, 0)),
                  pl.BlockSpec((BLK, 128), lambda i: (i, 0)),
                  pl.BlockSpec((BLK, 128), lambda i: (i, 0)),
                  pl.BlockSpec((BLK, 128), lambda i: (i, 0)),
                  pl.BlockSpec((BLK, 1), lambda i: (i, 0)),
                  pl.BlockSpec((1, 256), lambda i: (0, 0)),
                  pl.BlockSpec((256, 128), lambda i: (0, 0))],
        out_specs=[pl.BlockSpec((BLK, 128), lambda i: (i, 0))],
        out_shape=[jax.ShapeDtypeStruct((NMM, 128), _f32)],
    )


def _final_body(agg_ref, hs_ref, dinv_ref, b_ref, out_ref):
    out_ref[...] = ((agg_ref[...] + hs_ref[...]) * dinv_ref[...]
                    + b_ref[...])


@functools.cache
def _final():
    return pl.pallas_call(
        _final_body,
        grid=(NMM // BLK,),
        in_specs=[pl.BlockSpec((BLK, 128), lambda i: (i, 0)),
                  pl.BlockSpec((BLK, 128), lambda i: (i, 0)),
                  pl.BlockSpec((BLK, 1), lambda i: (i, 0)),
                  pl.BlockSpec((1, 128), lambda i: (0, 0))],
        out_specs=pl.BlockSpec((BLK, 128), lambda i: (i, 0)),
        out_shape=jax.ShapeDtypeStruct((NMM, 128), _f32),
    )


def kernel(x, edge_index, W1, b1, W2, b2, W3, b3, W4, b4):
    src = edge_index[0].astype(_i32)
    dst = edge_index[1].astype(_i32)

    counts = _a1()(dst).reshape(NW, 16, L)[:, :NSLAB, :].sum(-1)
    cpad = ((counts + 15) // 8) * 8
    tot = cpad.sum(axis=0)
    ss = jnp.concatenate([jnp.zeros((1,), _i32),
                          jnp.cumsum(tot).astype(_i32)])
    woff = ss[None, :NSLAB] + (jnp.cumsum(cpad, axis=0) - cpad)
    woff16 = jnp.zeros((NW, L), _i32).at[:, :NSLAB].set(woff)
    sb16 = jnp.zeros((L,), _i32).at[:NSLAB + 1].set(ss)

    esrc, edstl = _a2()(src, dst, woff16)
    deg = _deg()(esrc, edstl, sb16)

    xp = jnp.zeros((NMM, x.shape[1]), _f32).at[:N].set(x)
    W1p = jnp.zeros((x.shape[1], 128), _f32).at[:, :64].set(W1)
    b1p = jnp.zeros((128,), _f32).at[:64].set(b1)
    W2p = jnp.zeros((128, 128), _f32).at[:64].set(W2)
    hs1, dinv = _mm1(x.shape[1], 128)(xp, W1p, deg[:NMM, None])
    agg1 = _agg(128)(hs1, esrc, edstl, sb16)
    hs2, = _mid(128, 128, 1)(agg1[:NMM], hs1, dinv, b1p[None], W2p)
    agg2 = _agg(128)(hs2, esrc, edstl, sb16)
    hs3a, hs3b = _mid(128, 256, 2)(agg2[:NMM], hs2, dinv, b2[None], W3)
    agg3a = _agg(128)(hs3a, esrc, edstl, sb16)
    agg3b = _agg(128)(hs3b, esrc, edstl, sb16)
    W4p = jnp.zeros((256, 128), _f32).at[:, :3].set(W4)
    b4p = jnp.zeros((128,), _f32).at[:3].set(b4)
    hs4, = _mm4()(agg3a[:NMM], agg3b[:NMM], hs3a, hs3b, dinv, b3[None], W4p)
    agg4 = _agg(128)(hs4, esrc, edstl, sb16)
    outp = _final()(agg4[:NMM], hs4, dinv, b4p[None])
    return outp[:N, :3]


# speedup vs baseline: 6.6653x; 1.0036x over previous
import functools

import jax
import jax.numpy as jnp
from jax import lax
from jax.experimental import pallas as pl
from jax.experimental.pallas import tpu as pltpu
from jax.experimental.pallas import tpu_sc as plsc

N = 100000
E = 1600000
NC, NS, L = 2, 16, 16
NW = NC * NS
EPW = E // NW
CH = 2000
NCH = EPW // CH
VPC = CH // L
SLAB = 8192
SLAB_SHIFT = 13
NSLAB = 13
NPAD = SLAB * NSLAB
BLK = 2048
NMM = 100352
ZR = N
EPALLOC = E + NW * NSLAB * 16 + 128
STG = 160
DEGPAD = SLAB + 128
ZBLK = SLAB // 128 // NS

_i32 = jnp.int32
_f32 = jnp.float32


@functools.cache
def _mesh():
    return plsc.VectorSubcoreMesh(core_axis_name="c", subcore_axis_name="s",
                                  num_cores=NC, num_subcores=NS)


def _lanes():
    return lax.iota(_i32, L)


def _count_body(dst_hbm, counts_hbm, dchunk, cmat):
    w = lax.axis_index("s") * NC + lax.axis_index("c")
    base = w * EPW

    def chunk(i, carry):
        pltpu.sync_copy(dst_hbm.at[pl.ds(pl.multiple_of(base + i * CH, 8), CH)], dchunk)

        def vreg(j, carry):
            sid = lax.shift_right_logical(dchunk[pl.ds(j * L, L)], SLAB_SHIFT)
            return tuple(carry[s] + jnp.where(sid == s, 1, 0)
                         for s in range(NSLAB))

        return lax.fori_loop(0, VPC, vreg, carry)

    carry = lax.fori_loop(0, NCH, chunk,
                          (jnp.zeros((L,), _i32),) * NSLAB)
    for s in range(NSLAB):
        cmat[pl.ds(s * L, L)] = carry[s]
    for s in range(NSLAB, 16):
        cmat[pl.ds(s * L, L)] = jnp.zeros((L,), _i32)
    pltpu.sync_copy(cmat, counts_hbm.at[w])


@functools.cache
def _a1():
    return pl.kernel(
        _count_body,
        out_type=jax.ShapeDtypeStruct((NW, 16 * L), _i32),
        mesh=_mesh(),
        scratch_types=[pltpu.VMEM((CH,), _i32), pltpu.VMEM((16 * L,), _i32)],
    )


def _dist_body(src_hbm, dst_hbm, woff_hbm, esrc, edstl,
               schunk0, schunk1, dchunk0, dchunk1, tbufa0, tbufa1,
               tbufb0, tbufb1, gbuf, sent_s, sent_d, wbuf, posbuf,
               ld0, ld1, sc0, sc1):
    base = (lax.axis_index("s") * NC + lax.axis_index("c")) * EPW
    pltpu.sync_copy(woff_hbm.at[lax.axis_index("s") * NC
                                + lax.axis_index("c")], wbuf)
    wo_vec = wbuf[...]
    lanes = _lanes()
    schunks, dchunks = (schunk0, schunk1), (dchunk0, dchunk1)
    tbufas, tbufbs = (tbufa0, tbufa1), (tbufb0, tbufb1)
    lds, scs = (ld0, ld1), (sc0, sc1)
    sent_s[pl.ds(0, L)] = jnp.full((L,), ZR, _i32)
    sent_d[pl.ds(0, L)] = jnp.zeros((L,), _i32)
    posbuf[pl.ds(0, L)] = jnp.zeros((L,), _i32)

    def issue_loads(i, b):
        off = pl.multiple_of(base + i * CH, 8)
        pltpu.async_copy(src_hbm.at[pl.ds(off, CH)], schunks[b], lds[b])
        pltpu.async_copy(dst_hbm.at[pl.ds(off, CH)], dchunks[b], lds[b])

    def fire_or_drain(b, drain):
        def dmarow(r, u):
            off_r = pl.multiple_of(r * 128, 8)
            for args in ((schunks[b].at[pl.ds(off_r, 128)],
                          esrc.at[tbufas[b].at[r]], scs[b]),
                         (dchunks[b].at[pl.ds(off_r, 128)],
                          edstl.at[tbufas[b].at[r]], scs[b])):
                if drain:
                    pltpu.make_async_copy(*args).wait()
                else:
                    pltpu.async_copy(*args)
            return u

        lax.fori_loop(0, 15, dmarow, jnp.zeros((), _i32))
        for args in ((schunks[b].at[pl.ds(1920, 80)],
                      esrc.at[tbufbs[b].at[0]], scs[b]),
                     (dchunks[b].at[pl.ds(1920, 80)],
                      edstl.at[tbufbs[b].at[0]], scs[b])):
            if drain:
                pltpu.make_async_copy(*args).wait()
            else:
                pltpu.async_copy(*args)

    def vreg(j, posv, b, store):
        sv = schunks[b][pl.ds(j * L, L)]
        dv = dchunks[b][pl.ds(j * L, L)]
        sid = lax.shift_right_logical(dv, SLAB_SHIFT)
        dloc = jnp.bitwise_and(dv, SLAB - 1)
        dchunks[b][pl.ds(j * L, L)] = dloc
        rank = jnp.zeros((L,), _i32)
        for dd in range(1, L):
            sh = sid.at[jnp.maximum(lanes - dd, 0)].get(
                mode="promise_in_bounds")
            rank = rank + jnp.where((lanes >= dd) & (sh == sid), 1, 0)
        tgt = (wo_vec + posv).at[sid].get(mode="promise_in_bounds") + rank
        store(tgt)
        for l in range(L):
            posv = posv + jnp.where(lanes == sid[l], 1, 0)
        return posv

    issue_loads(jnp.zeros((), _i32), 0)

    def step(i2, u):
        for b in (0, 1):
            i = 2 * i2 + b

            @pl.when(i < NCH)
            def _(b=b, i=i):
                pltpu.make_async_copy(src_hbm.at[pl.ds(0, CH)],
                                      schunks[b], lds[b]).wait()
                pltpu.make_async_copy(src_hbm.at[pl.ds(0, CH)],
                                      dchunks[b], lds[b]).wait()
                posv0 = posbuf[pl.ds(0, L)]

                def row(r, posv):
                    for jj in range(8):
                        def store(tgt, r=r, jj=jj, b=b):
                            tbufas[b][r, pl.ds(jj * L, L)] = tgt
                        posv = vreg(r * 8 + jj, posv, b, store)
                    return posv

                posv = lax.fori_loop(0, 15, row, posv0)
                for jj in range(5):
                    def store(tgt, jj=jj, b=b):
                        tbufbs[b][0, pl.ds(jj * L, L)] = tgt
                    posv = vreg(120 + jj, posv, b, store)
                posbuf[pl.ds(0, L)] = posv

                @pl.when(i >= 1)
                def _():
                    fire_or_drain(b ^ 1, True)

                @pl.when(i + 1 < NCH)
                def _():
                    issue_loads(i + 1, b ^ 1)

                fire_or_drain(b, False)
        return u

    lax.fori_loop(0, (NCH + 1) // 2, step, jnp.zeros((), _i32))
    fire_or_drain(0, True)

    posv = posbuf[pl.ds(0, L)]
    for s in range(NSLAB):
        cnt = posv[s]
        cpad = lax.shift_left(lax.shift_right_logical(cnt + 15, 3), 3)
        wo_s = wo_vec[s]
        gbuf[0, pl.ds(0, L)] = wo_s + jnp.minimum(cnt + lanes, cpad - 1)
        pltpu.sync_copy(sent_s, esrc.at[gbuf.at[0]])
        pltpu.sync_copy(sent_d, edstl.at[gbuf.at[0]])


@functools.cache
def _a2():
    return pl.kernel(
        _dist_body,
        out_type=(jax.ShapeDtypeStruct((EPALLOC,), _i32),
                  jax.ShapeDtypeStruct((EPALLOC,), _i32)),
        mesh=_mesh(),
        scratch_types=[
            pltpu.VMEM((CH,), _i32), pltpu.VMEM((CH,), _i32),
            pltpu.VMEM((CH,), _i32), pltpu.VMEM((CH,), _i32),
            pltpu.VMEM((15, 128), _i32), pltpu.VMEM((15, 128), _i32),
            pltpu.VMEM((1, 80), _i32), pltpu.VMEM((1, 80), _i32),
            pltpu.VMEM((1, L), _i32),
            pltpu.VMEM((L,), _i32), pltpu.VMEM((L,), _i32),
            pltpu.VMEM((L,), _i32), pltpu.VMEM((L,), _i32),
            pltpu.SemaphoreType.DMA, pltpu.SemaphoreType.DMA,
            pltpu.SemaphoreType.DMA, pltpu.SemaphoreType.DMA,
        ],
    )


def _deg_body(esrc, edstl, sb_hbm, deg_hbm,
              sbuf, dbuf, ones_v, zbuf, sbvec, acc1):
    c = lax.axis_index("c")
    t = lax.axis_index("s")
    lanes = _lanes()
    pltpu.sync_copy(sb_hbm, sbvec)
    sb = sbvec[...]

    def vfill(k, u):
        ones_v[pl.ds(k * L, L)] = jnp.ones((L,), _f32)
        zbuf[pl.ds(k * L, L)] = jnp.zeros((L,), _f32)
        return u

    lax.fori_loop(0, 128 // L, vfill, jnp.zeros((), _i32))

    for si in range(7):
        s = 2 * si + c

        def do_slab(s=s):
            nblkz = DEGPAD // 128

            def zblk(i, u):
                b = t + i * NS

                @pl.when(b < nblkz)
                def _():
                    pltpu.sync_copy(zbuf, acc1.at[pl.ds(pl.multiple_of(b * 128, 8), 128)])

                return u

            lax.fori_loop(0, (nblkz + NS - 1) // NS, zblk,
                          jnp.zeros((), _i32))
            plsc.subcore_barrier()

            rs = jnp.where(c == 0, sb[2 * si], sb[2 * si + 1])
            re = jnp.where(c == 0, sb[2 * si + 1], sb[2 * si + 2])
            nch = lax.shift_right_logical(re - rs + 127, 7)
            nt = jnp.maximum(nch - t + NS - 1, 0) // NS

            def chunk(i, u):
                c0 = rs + (t + i * NS) * 128
                pltpu.sync_copy(esrc.at[pl.ds(pl.multiple_of(c0, 8), 128)], sbuf)
                pltpu.sync_copy(edstl.at[pl.ds(pl.multiple_of(c0, 8), 128)], dbuf)
                for j in range(128 // L):
                    valid = (c0 + j * L + lanes) < re
                    sv = sbuf[pl.ds(j * L, L)]
                    dv = dbuf[pl.ds(j * L, L)]
                    keep = jnp.logical_and(valid, sv != ZR)
                    dbuf[pl.ds(j * L, L)] = jnp.where(keep, dv, SLAB)
                pltpu.sync_copy(ones_v, acc1.at[dbuf], add=True)
                return u

            lax.fori_loop(0, nt, chunk, jnp.zeros((), _i32))
            plsc.subcore_barrier()
            for k in range(ZBLK):
                b = t * ZBLK + k
                pltpu.sync_copy(acc1.at[pl.ds(pl.multiple_of(b * 128, 8), 128)],
                                deg_hbm.at[pl.ds(pl.multiple_of(s * SLAB + b * 128, 8), 128)])
            plsc.subcore_barrier()

        if si < 6:
            do_slab()
        else:
            pl.when(c == 0)(do_slab)


@functools.cache
def _deg():
    return pl.kernel(
        _deg_body,
        out_type=jax.ShapeDtypeStruct((NPAD,), _f32),
        mesh=_mesh(),
        scratch_types=[
            pltpu.VMEM((128,), _i32), pltpu.VMEM((128,), _i32),
            pltpu.VMEM((128,), _f32), pltpu.VMEM((128,), _f32),
            pltpu.VMEM((L,), _i32),
            pltpu.VMEM_SHARED((DEGPAD,), _f32),
        ],
    )


def _agg_body(d, hs, esrc, edstl, sb_hbm, agg_hbm,
              sbuf0, sbuf1, dbuf0, dbuf1, rows0, rows1, zbuf, sbvec,
              ld0, ld1, g0, g1, sc0, sc1, acc):
    c = lax.axis_index("c")
    t = lax.axis_index("s")
    lanes = _lanes()
    sbufs, dbufs = (sbuf0, sbuf1), (dbuf0, dbuf1)
    rows, lds, gs, scs = (rows0, rows1), (ld0, ld1), (g0, g1), (sc0, sc1)
    pltpu.sync_copy(sb_hbm, sbvec)
    sb = sbvec[...]

    def zrow(i, u):
        def zcol(j, u):
            zbuf[i, pl.ds(j * L, L)] = jnp.zeros((L,), _f32)
            return u

        return lax.fori_loop(0, d // L, zcol, u)

    lax.fori_loop(0, 128, zrow, jnp.zeros((), _i32))

    for si in range(7):
        s = 2 * si + c

        def do_slab(s=s, si=si):
            for k in range(ZBLK):
                b = t * ZBLK + k
                pltpu.sync_copy(zbuf, acc.at[pl.ds(pl.multiple_of(b * 128, 8), 128)])
            plsc.subcore_barrier()

            rs = jnp.where(c == 0, sb[2 * si], sb[2 * si + 1])
            re = jnp.where(c == 0, sb[2 * si + 1], sb[2 * si + 2])
            nch = lax.shift_right_logical(re - rs + 127, 7)
            nt = jnp.maximum(nch - t + NS - 1, 0) // NS

            def issue_loads(k, b):
                c0 = pl.multiple_of(rs + (t + k * NS) * 128, 8)
                pltpu.async_copy(esrc.at[pl.ds(c0, 128)], sbufs[b], lds[b])
                pltpu.async_copy(edstl.at[pl.ds(c0, 128)], dbufs[b], lds[b])

            @pl.when(nt > 0)
            def _():
                issue_loads(jnp.zeros((), _i32), 0)

            def step(i2, u):
                for b in (0, 1):
                    k = 2 * i2 + b

                    @pl.when(k < nt)
                    def _(b=b, k=k):
                        pltpu.make_async_copy(
                            esrc.at[pl.ds(0, 128)], sbufs[b], lds[b]).wait()
                        pltpu.make_async_copy(
                            esrc.at[pl.ds(0, 128)], dbufs[b], lds[b]).wait()
                        c0 = rs + (t + k * NS) * 128
                        for j in range(128 // L):
                            valid = (c0 + j * L + lanes) < re
                            sv = sbufs[b][pl.ds(j * L, L)]
                            dv = dbufs[b][pl.ds(j * L, L)]
                            sbufs[b][pl.ds(j * L, L)] = jnp.where(valid, sv, ZR)
                            dbufs[b][pl.ds(j * L, L)] = jnp.where(valid, dv, 0)

                        @pl.when(k + 1 < nt)
                        def _():
                            issue_loads(k + 1, b ^ 1)

                        @pl.when(k >= 2)
                        def _():
                            pltpu.make_async_copy(
                                rows[b], acc.at[dbufs[b]], scs[b]).wait()

                        pltpu.async_copy(hs.at[sbufs[b]], rows[b], gs[b])
                        pltpu.make_async_copy(hs.at[sbufs[b]], rows[b],
                                              gs[b]).wait()
                        pltpu.async_copy(rows[b], acc.at[dbufs[b]], scs[b],
                                         add=True)
                return u

            lax.fori_loop(0, (nt + 1) // 2, step, jnp.zeros((), _i32))
            for b in (0, 1):
                @pl.when(nt > b)
                def _(b=b):
                    pltpu.make_async_copy(rows[b], acc.at[dbufs[b]],
                                          scs[b]).wait()
            plsc.subcore_barrier()
            for k in range(ZBLK):
                b = t * ZBLK + k
                pltpu.sync_copy(acc.at[pl.ds(pl.multiple_of(b * 128, 8), 128)],
                                agg_hbm.at[pl.ds(pl.multiple_of(s * SLAB + b * 128, 8), 128)])
            plsc.subcore_barrier()

        if si < 6:
            do_slab()
        else:
            pl.when(c == 0)(do_slab)


@functools.cache
def _agg(d):
    return pl.kernel(
        functools.partial(_agg_body, d),
        out_type=jax.ShapeDtypeStruct((NPAD, d), _f32),
        mesh=_mesh(),
        scratch_types=[
            pltpu.VMEM((128,), _i32), pltpu.VMEM((128,), _i32),
            pltpu.VMEM((128,), _i32), pltpu.VMEM((128,), _i32),
            pltpu.VMEM((128, d), _f32), pltpu.VMEM((128, d), _f32),
            pltpu.VMEM((128, d), _f32),
            pltpu.VMEM((L,), _i32),
            pltpu.SemaphoreType.DMA, pltpu.SemaphoreType.DMA,
            pltpu.SemaphoreType.DMA, pltpu.SemaphoreType.DMA,
            pltpu.SemaphoreType.DMA, pltpu.SemaphoreType.DMA,
            pltpu.VMEM_SHARED((SLAB, d), _f32),
        ],
    )


def _mm1_body(x_ref, w_ref, deg_ref, hs_ref, dinv_ref):
    dv = lax.rsqrt(deg_ref[...] + 1.0)
    hs_ref[...] = jnp.dot(x_ref[...], w_ref[...],
                          preferred_element_type=_f32) * dv
    dinv_ref[...] = dv


@functools.cache
def _mm1(din, dout):
    return pl.pallas_call(
        _mm1_body,
        grid=(NMM // BLK,),
        in_specs=[pl.BlockSpec((BLK, din), lambda i: (i, 0)),
                  pl.BlockSpec((din, dout), lambda i: (0, 0)),
                  pl.BlockSpec((BLK, 1), lambda i: (i, 0))],
        out_specs=[pl.BlockSpec((BLK, dout), lambda i: (i, 0)),
                   pl.BlockSpec((BLK, 1), lambda i: (i, 0))],
        out_shape=[jax.ShapeDtypeStruct((NMM, dout), _f32),
                   jax.ShapeDtypeStruct((NMM, 1), _f32)],
    )


def _mid_body(nout, agg_ref, hs_ref, dinv_ref, b_ref, w_ref, *out_refs):
    dv = dinv_ref[...]
    tv = jnp.maximum((agg_ref[...] + hs_ref[...]) * dv + b_ref[...], 0.0)
    w = w_ref[...]
    step = w.shape[1] // nout
    for k, o_ref in enumerate(out_refs):
        o_ref[...] = jnp.dot(tv, w[:, k * step:(k + 1) * step],
                             preferred_element_type=_f32) * dv


@functools.cache
def _mid(dprev, dnext, nout):
    step = dnext // nout
    return pl.pallas_call(
        functools.partial(_mid_body, nout),
        grid=(NMM // BLK,),
        in_specs=[pl.BlockSpec((BLK, dprev), lambda i: (i, 0)),
                  pl.BlockSpec((BLK, dprev), lambda i: (i, 0)),
                  pl.BlockSpec((BLK, 1), lambda i: (i, 0)),
                  pl.BlockSpec((1, dprev), lambda i: (0, 0)),
                  pl.BlockSpec((dprev, dnext), lambda i: (0, 0))],
        out_specs=[pl.BlockSpec((BLK, step), lambda i: (i, 0))] * nout,
        out_shape=[jax.ShapeDtypeStruct((NMM, step), _f32)] * nout,
    )


def _mm4_body(a1_ref, a2_ref, h1_ref, h2_ref, dinv_ref, b_ref, w_ref,
              out_ref):
    dv = dinv_ref[...]
    b = b_ref[...]
    t1 = jnp.maximum((a1_ref[...] + h1_ref[...]) * dv + b[:, :128], 0.0)
    t2 = jnp.maximum((a2_ref[...] + h2_ref[...]) * dv + b[:, 128:], 0.0)
    w = w_ref[...]
    out_ref[...] = (jnp.dot(t1, w[:128], preferred_element_type=_f32)
                    + jnp.dot(t2, w[128:], preferred_element_type=_f32)) * dv


@functools.cache
def _mm4():
    return pl.pallas_call(
        _mm4_body,
        grid=(NMM // BLK,),
        in_specs=[pl.BlockSpec((BLK, 128), lambda i: (i, 0)),
                  pl.BlockSpec((BLK, 128), lambda i: (i, 0)),
                  pl.BlockSpec((BLK, 128), lambda i: (i, 0)),
                  pl.BlockSpec((BLK, 128), lambda i: (i, 0)),
                  pl.BlockSpec((BLK, 1), lambda i: (i, 0)),
                  pl.BlockSpec((1, 256), lambda i: (0, 0)),
                  pl.BlockSpec((256, 128), lambda i: (0, 0))],
        out_specs=[pl.BlockSpec((BLK, 128), lambda i: (i, 0))],
        out_shape=[jax.ShapeDtypeStruct((NMM, 128), _f32)],
    )


def _final_body(agg_ref, hs_ref, dinv_ref, b_ref, out_ref):
    out_ref[...] = ((agg_ref[...] + hs_ref[...]) * dinv_ref[...]
                    + b_ref[...])


@functools.cache
def _final():
    return pl.pallas_call(
        _final_body,
        grid=(NMM // BLK,),
        in_specs=[pl.BlockSpec((BLK, 128), lambda i: (i, 0)),
                  pl.BlockSpec((BLK, 128), lambda i: (i, 0)),
                  pl.BlockSpec((BLK, 1), lambda i: (i, 0)),
                  pl.BlockSpec((1, 128), lambda i: (0, 0))],
        out_specs=pl.BlockSpec((BLK, 128), lambda i: (i, 0)),
        out_shape=jax.ShapeDtypeStruct((NMM, 128), _f32),
    )


def kernel(x, edge_index, W1, b1, W2, b2, W3, b3, W4, b4):
    src = edge_index[0].astype(_i32)
    dst = edge_index[1].astype(_i32)

    counts = _a1()(dst).reshape(NW, 16, L)[:, :NSLAB, :].sum(-1)
    cpad = ((counts + 15) // 8) * 8
    tot = cpad.sum(axis=0)
    ss = jnp.concatenate([jnp.zeros((1,), _i32),
                          jnp.cumsum(tot).astype(_i32)])
    woff = ss[None, :NSLAB] + (jnp.cumsum(cpad, axis=0) - cpad)
    woff16 = jnp.zeros((NW, L), _i32).at[:, :NSLAB].set(woff)
    sb16 = jnp.zeros((L,), _i32).at[:NSLAB + 1].set(ss)

    esrc, edstl = _a2()(src, dst, woff16)
    deg = _deg()(esrc, edstl, sb16)

    xp = jnp.zeros((NMM, x.shape[1]), _f32).at[:N].set(x)
    W1p = jnp.zeros((x.shape[1], 128), _f32).at[:, :64].set(W1)
    b1p = jnp.zeros((128,), _f32).at[:64].set(b1)
    W2p = jnp.zeros((128, 128), _f32).at[:64].set(W2)
    hs1, dinv = _mm1(x.shape[1], 128)(xp, W1p, deg[:NMM, None])
    agg1 = _agg(128)(hs1, esrc, edstl, sb16)
    hs2, = _mid(128, 128, 1)(agg1[:NMM], hs1, dinv, b1p[None], W2p)
    agg2 = _agg(128)(hs2, esrc, edstl, sb16)
    hs3a, hs3b = _mid(128, 256, 2)(agg2[:NMM], hs2, dinv, b2[None], W3)
    agg3a = _agg(128)(hs3a, esrc, edstl, sb16)
    agg3b = _agg(128)(hs3b, esrc, edstl, sb16)
    W4p = jnp.zeros((256, 128), _f32).at[:, :3].set(W4)
    b4p = jnp.zeros((128,), _f32).at[:3].set(b4)
    hs4, = _mm4()(agg3a[:NMM], agg3b[:NMM], hs3a, hs3b, dinv, b3[None], W4p)
    agg4 = _agg(128)(hs4, esrc, edstl, sb16)
    outp = _final()(agg4[:NMM], hs4, dinv, b4p[None])
    return outp[:N, :3]
